# Initial kernel scaffold; baseline (speedup 1.0000x reference)
import jax
import jax.numpy as jnp
from jax.experimental import pallas as pl


def kernel(index, table):
    raise NotImplementedError("write your pallas kernel here")


# speedup vs baseline: 1.0275x; 1.0275x over previous
import functools

import jax
import jax.numpy as jnp
from jax import lax
from jax.experimental import pallas as pl
from jax.experimental.pallas import tpu as pltpu
from jax.experimental.pallas import tpu_sc as plsc

_N = 1024 * 50
_V = 1000
_VP = 1024
_NW = 32
_BPW = _N // _NW
_C = 40
_NCHUNK = _BPW // _C

_mesh = plsc.VectorSubcoreMesh(core_axis_name="c", subcore_axis_name="s")


@functools.partial(
    pl.kernel,
    out_type=jax.ShapeDtypeStruct((_N, _V), jnp.float32),
    mesh=_mesh,
    scratch_types=[
        pltpu.VMEM((_BPW,), jnp.int32),
        pltpu.VMEM((2, _C, _V), jnp.float32),
        pltpu.SemaphoreType.DMA,
        pltpu.SemaphoreType.DMA,
        pltpu.SemaphoreType.DMA,
        pltpu.SemaphoreType.DMA,
    ],
    compiler_params=pltpu.CompilerParams(use_tc_tiling_on_sc=False),
)
def _gather(idx_hbm, table_hbm, out_hbm, idx_v, rows_v, gsem0, gsem1,
            osem0, osem1):
    wid = lax.axis_index("s") * 2 + lax.axis_index("c")
    base = wid * _BPW
    pltpu.sync_copy(idx_hbm.at[pl.ds(base, _BPW)], idx_v)

    def gather_desc(g, buf, sem):
        return pltpu.make_async_copy(
            table_hbm.at[idx_v.at[pl.ds(g * _C, _C)]], rows_v.at[buf], sem
        )

    def out_desc(g, buf, sem):
        return pltpu.make_async_copy(
            rows_v.at[buf], out_hbm.at[pl.ds(base + g * _C, _C)], sem
        )

    gather_desc(0, 0, gsem0).start()

    def outer(o, carry):
        g0 = o * 2
        gather_desc(g0, 0, gsem0).wait()

        @pl.when(o > 0)
        def _():
            out_desc(g0 - 1, 1, osem1).wait()

        gather_desc(g0 + 1, 1, gsem1).start()
        out_desc(g0, 0, osem0).start()

        gather_desc(g0 + 1, 1, gsem1).wait()

        @pl.when(o < _NCHUNK // 2 - 1)
        def _():
            out_desc(g0, 0, osem0).wait()
            gather_desc(g0 + 2, 0, gsem0).start()

        out_desc(g0 + 1, 1, osem1).start()
        return carry

    lax.fori_loop(0, _NCHUNK // 2, outer, 0)
    out_desc(_NCHUNK - 2, 0, osem0).wait()
    out_desc(_NCHUNK - 1, 1, osem1).wait()


def kernel(index, table):
    idx = index.reshape(-1)
    out = _gather(idx, table)
    return out.reshape(index.shape[0], index.shape[1], _V)


# speedup vs baseline: 1.0302x; 1.0026x over previous
import functools

import jax
import jax.numpy as jnp
from jax import lax
from jax.experimental import pallas as pl
from jax.experimental.pallas import tpu as pltpu
from jax.experimental.pallas import tpu_sc as plsc

_B = 1024
_T = 50
_V = 1000
_NW = 32
_BPW = _B // _NW
_mesh = plsc.VectorSubcoreMesh(core_axis_name="c", subcore_axis_name="s")


@functools.partial(
    pl.kernel,
    out_type=jax.ShapeDtypeStruct((_B, _T, _V), jnp.float32),
    mesh=_mesh,
    scratch_types=[
        pltpu.VMEM((_BPW, _T), jnp.int32),
        pltpu.VMEM((2, _T, _V), jnp.float32),
        pltpu.SemaphoreType.DMA,
        pltpu.SemaphoreType.DMA,
        pltpu.SemaphoreType.DMA,
        pltpu.SemaphoreType.DMA,
    ],
    compiler_params=pltpu.CompilerParams(use_tc_tiling_on_sc=False),
)
def _gather(idx_hbm, table_hbm, out_hbm, idx_v, rows_v, gsem0, gsem1,
            osem0, osem1):
    wid = lax.axis_index("s") * 2 + lax.axis_index("c")
    bb = wid * _BPW
    pltpu.sync_copy(idx_hbm.at[pl.ds(bb, _BPW)], idx_v)

    def gather_desc(g, buf, sem):
        return pltpu.make_async_copy(
            table_hbm.at[idx_v.at[g]], rows_v.at[buf], sem
        )

    def out_desc(g, buf, sem):
        return pltpu.make_async_copy(rows_v.at[buf], out_hbm.at[bb + g], sem)

    gather_desc(0, 0, gsem0).start()

    def outer(o, carry):
        g0 = o * 2
        gather_desc(g0, 0, gsem0).wait()

        @pl.when(o > 0)
        def _():
            out_desc(g0 - 1, 1, osem1).wait()

        gather_desc(g0 + 1, 1, gsem1).start()
        out_desc(g0, 0, osem0).start()

        gather_desc(g0 + 1, 1, gsem1).wait()

        @pl.when(o < _BPW // 2 - 1)
        def _():
            out_desc(g0, 0, osem0).wait()
            gather_desc(g0 + 2, 0, gsem0).start()

        out_desc(g0 + 1, 1, osem1).start()
        return carry

    lax.fori_loop(0, _BPW // 2, outer, 0)
    out_desc(_BPW - 2, 0, osem0).wait()
    out_desc(_BPW - 1, 1, osem1).wait()


def kernel(index, table):
    return _gather(index, table)
